# Initial kernel scaffold; baseline (speedup 1.0000x reference)
import jax
import jax.numpy as jnp
from jax.experimental import pallas as pl


def kernel(xyz, params):
    raise NotImplementedError("write your pallas kernel here")


# speedup vs baseline: 6.7104x; 6.7104x over previous
import jax
import jax.numpy as jnp
from jax.experimental import pallas as pl

_EPS = 1e-5


def _fold(layer):
    s = layer['g'] / jnp.sqrt(layer['rv'] + _EPS)
    Wt = (layer['W'] * s[:, None]).T
    b = s * (layer['b'] - layer['rm']) + layer['be']
    return Wt, b


def _pad_cols(x, m=8):
    c = x.shape[-1]
    p = (-c) % m
    if p == 0:
        return x
    return jnp.pad(x, [(0, 0)] * (x.ndim - 1) + [(0, p)])


def _pad_rows(w, m=8):
    r = w.shape[0]
    p = (-r) % m
    if p == 0:
        return w
    return jnp.pad(w, [(0, p), (0, 0)])


def _mlp(x, layers, relu_flags, tile=512):
    N = x.shape[0]
    tile = min(tile, N)
    nl = len(layers)

    flat = []
    in_specs = [pl.BlockSpec((tile, x.shape[1]), lambda i: (i, 0))]
    for Wt, b in layers:
        b2 = jnp.tile(b[None, :], (8, 1))
        in_specs.append(pl.BlockSpec(Wt.shape, lambda i: (0, 0)))
        in_specs.append(pl.BlockSpec(b2.shape, lambda i: (0, 0)))
        flat += [Wt, b2]
    Co = layers[-1][0].shape[1]

    def body(x_ref, *refs):
        out_ref = refs[-1]
        h = x_ref[...]
        for i in range(nl):
            W = refs[2 * i][...]
            b = refs[2 * i + 1][0:1, :]
            h = jnp.dot(h, W, preferred_element_type=jnp.float32) + b
            if relu_flags[i]:
                h = jnp.maximum(h, 0.0)
        out_ref[...] = h

    return pl.pallas_call(
        body,
        grid=(N // tile,),
        in_specs=in_specs,
        out_specs=pl.BlockSpec((tile, Co), lambda i: (i, 0)),
        out_shape=jax.ShapeDtypeStruct((N, Co), jnp.float32),
    )(x, *flat)


def _knnmax(cxyz, pxyzT, y, k, mt):
    B, M, _ = cxyz.shape
    P = pxyzT.shape[2]
    C = y.shape[2]
    big = float('inf')

    def body(c_ref, pT_ref, y_ref, out_ref):
        c = c_ref[0]
        pT = pT_ref[0]
        yv = y_ref[0]
        aa = jnp.sum(c * c, axis=1, keepdims=True)
        bb = jnp.sum(pT * pT, axis=0, keepdims=True)
        d2 = aa + bb - 2.0 * jnp.dot(c, pT, preferred_element_type=jnp.float32)
        iota = jax.lax.broadcasted_iota(jnp.int32, (mt, P), 1)

        def step(_, carry):
            d2c, pooled = carry
            m = jnp.min(d2c, axis=1, keepdims=True)
            amin = jnp.min(jnp.where(d2c == m, iota, P), axis=1, keepdims=True)
            onehot = iota == amin
            g = jnp.dot(onehot.astype(jnp.float32), yv,
                        preferred_element_type=jnp.float32)
            pooled = jnp.maximum(pooled, g)
            d2c = jnp.where(onehot, big, d2c)
            return d2c, pooled

        pooled0 = jnp.full((mt, C), -big, jnp.float32)
        _, pooled = jax.lax.fori_loop(0, k, step, (d2, pooled0))
        out_ref[0] = pooled

    return pl.pallas_call(
        body,
        grid=(B, M // mt),
        in_specs=[
            pl.BlockSpec((1, mt, 8), lambda b, i: (b, i, 0)),
            pl.BlockSpec((1, 8, P), lambda b, i: (b, 0, 0)),
            pl.BlockSpec((1, P, C), lambda b, i: (b, 0, 0)),
        ],
        out_specs=pl.BlockSpec((1, mt, C), lambda b, i: (b, i, 0)),
        out_shape=jax.ShapeDtypeStruct((B, M, C), jnp.float32),
    )(cxyz, pxyzT, y)


def _interp(x1, x2T, f2, nt):
    B, N1, _ = x1.shape
    P2 = x2T.shape[2]
    C2 = f2.shape[2]
    big = float('inf')

    def body(x1_ref, x2T_ref, f2_ref, out_ref):
        c = x1_ref[0]
        pT = x2T_ref[0]
        fv = f2_ref[0]
        aa = jnp.sum(c * c, axis=1, keepdims=True)
        bb = jnp.sum(pT * pT, axis=0, keepdims=True)
        d2 = aa + bb - 2.0 * jnp.dot(c, pT, preferred_element_type=jnp.float32)
        d2 = jnp.maximum(d2, 0.0)
        d = jnp.where(d2 > 1e-12, jnp.sqrt(jnp.where(d2 > 1e-12, d2, 1.0)), 0.0)
        d = jnp.maximum(d, 1e-8)
        iota = jax.lax.broadcasted_iota(jnp.int32, (nt, P2), 1)
        dwork = d
        sel = jnp.zeros((nt, P2), jnp.bool_)
        for _ in range(3):
            m = jnp.min(dwork, axis=1, keepdims=True)
            amin = jnp.min(jnp.where(dwork == m, iota, P2), axis=1,
                           keepdims=True)
            onehot = iota == amin
            sel = jnp.logical_or(sel, onehot)
            dwork = jnp.where(onehot, big, dwork)
        w = jnp.where(sel, 1.0 / d, 0.0)
        w = w / jnp.sum(w, axis=1, keepdims=True)
        out_ref[0] = jnp.dot(w, fv, preferred_element_type=jnp.float32)

    return pl.pallas_call(
        body,
        grid=(B, N1 // nt),
        in_specs=[
            pl.BlockSpec((1, nt, 8), lambda b, i: (b, i, 0)),
            pl.BlockSpec((1, 8, P2), lambda b, i: (b, 0, 0)),
            pl.BlockSpec((1, P2, C2), lambda b, i: (b, 0, 0)),
        ],
        out_specs=pl.BlockSpec((1, nt, C2), lambda b, i: (b, i, 0)),
        out_shape=jax.ShapeDtypeStruct((B, N1, C2), jnp.float32),
    )(x1, x2T, f2)


def _centers_idx(P):
    M = max(1, P // 4)
    return jnp.linspace(0.0, P - 1, M).astype(jnp.int32)


def _prep_mlp(layers, cin_padded):
    out = []
    for i, l in enumerate(layers):
        Wt, b = _fold(l)
        if i == 0:
            Wt = _pad_rows(Wt) if Wt.shape[0] != cin_padded else Wt
        out.append((Wt, b))
    return out


def kernel(xyz, params):
    B, P0, _ = xyz.shape
    f32 = jnp.float32
    xyz = xyz.astype(f32)

    xyz_p = _pad_cols(xyz)
    xyzT = jnp.transpose(xyz_p, (0, 2, 1))

    sa1 = _prep_mlp(params['sa1'], 8)
    Y1 = _mlp(xyz_p.reshape(B * P0, 8), sa1, [True] * 3)
    Y1 = Y1.reshape(B, P0, -1)
    idx1 = _centers_idx(P0)
    c1 = jnp.take(xyz_p, idx1, axis=1)
    l1 = _knnmax(c1, xyzT, Y1, k=32, mt=256)

    P1 = c1.shape[1]
    c1T = jnp.transpose(c1, (0, 2, 1))
    in2 = _pad_cols(jnp.concatenate([c1[..., :3], l1], axis=-1))
    sa2 = _prep_mlp(params['sa2'], in2.shape[-1])
    Y2 = _mlp(in2.reshape(B * P1, -1), sa2, [True] * 3).reshape(B, P1, -1)
    idx2 = _centers_idx(P1)
    c2 = jnp.take(c1, idx2, axis=1)
    l2 = _knnmax(c2, c1T, Y2, k=64, mt=256)

    P2 = c2.shape[1]
    c2T = jnp.transpose(c2, (0, 2, 1))
    in3 = _pad_cols(jnp.concatenate([c2[..., :3], l2], axis=-1))
    sa3 = _prep_mlp(params['sa3'], in3.shape[-1])
    Y3 = _mlp(in3.reshape(B * P2, -1), sa3, [True] * 3).reshape(B, P2, -1)
    idx3 = _centers_idx(P2)
    c3 = jnp.take(c2, idx3, axis=1)
    c3T = jnp.transpose(c3, (0, 2, 1))
    l3 = _knnmax(c3, c2T, Y3, k=128, mt=64)

    i3 = _interp(c2, c3T, l3, nt=256)
    fp3 = _prep_mlp(params['fp3'], 1280)
    cat3 = jnp.concatenate([i3, l2], axis=-1).reshape(B * P2, -1)
    l2n = _mlp(cat3, fp3, [True] * 2).reshape(B, P2, -1)

    i2 = _interp(c1, c2T, l2n, nt=512)
    fp2 = _prep_mlp(params['fp2'], 384)
    cat2 = jnp.concatenate([i2, l1], axis=-1).reshape(B * P1, -1)
    l1n = _mlp(cat2, fp2, [True] * 2).reshape(B, P1, -1)

    i1 = _interp(xyz_p, c1T, l1n, nt=512)
    fp1 = _prep_mlp(params['fp1'], 128)
    l0n = _mlp(i1.reshape(B * P0, -1), fp1, [True] * 3)

    h = params['head']
    h1 = _fold({'W': h['c1W'], 'b': h['c1b'], 'g': h['g'], 'be': h['be'],
                'rm': h['rm'], 'rv': h['rv']})
    h2 = (h['c2W'].T, h['c2b'])
    out = _mlp(l0n, [h1, h2], [True, False])
    return out.reshape(B, P0, -1)


# speedup vs baseline: 7.9742x; 1.1883x over previous
import functools

import jax
import jax.numpy as jnp
from jax import lax
from jax.experimental import pallas as pl
from jax.experimental.pallas import tpu as pltpu
from jax.experimental.pallas import tpu_sc as plsc

_EPS = 1e-5


def _fold(layer):
    s = layer['g'] / jnp.sqrt(layer['rv'] + _EPS)
    Wt = (layer['W'] * s[:, None]).T
    b = s * (layer['b'] - layer['rm']) + layer['be']
    return Wt, b


def _pad_cols(x, m=8):
    c = x.shape[-1]
    p = (-c) % m
    if p == 0:
        return x
    return jnp.pad(x, [(0, 0)] * (x.ndim - 1) + [(0, p)])


def _pad_rows(w, m=8):
    r = w.shape[0]
    p = (-r) % m
    if p == 0:
        return w
    return jnp.pad(w, [(0, p), (0, 0)])


def _mlp(x, layers, relu_flags, tile=512):
    N = x.shape[0]
    tile = min(tile, N)
    nl = len(layers)

    flat = []
    in_specs = [pl.BlockSpec((tile, x.shape[1]), lambda i: (i, 0))]
    for Wt, b in layers:
        b2 = jnp.tile(b[None, :], (8, 1))
        in_specs.append(pl.BlockSpec(Wt.shape, lambda i: (0, 0)))
        in_specs.append(pl.BlockSpec(b2.shape, lambda i: (0, 0)))
        flat += [Wt, b2]
    Co = layers[-1][0].shape[1]

    def body(x_ref, *refs):
        out_ref = refs[-1]
        h = x_ref[...]
        for i in range(nl):
            W = refs[2 * i][...]
            b = refs[2 * i + 1][0:1, :]
            h = jnp.dot(h, W, preferred_element_type=jnp.float32) + b
            if relu_flags[i]:
                h = jnp.maximum(h, 0.0)
        out_ref[...] = h

    return pl.pallas_call(
        body,
        grid=(N // tile,),
        in_specs=in_specs,
        out_specs=pl.BlockSpec((tile, Co), lambda i: (i, 0)),
        out_shape=jax.ShapeDtypeStruct((N, Co), jnp.float32),
    )(x, *flat)


def _knn_idx(cxyz, pxyzT, k, mt):
    B, M, _ = cxyz.shape
    P = pxyzT.shape[2]
    big = float('inf')

    def body(c_ref, pT_ref, out_ref):
        b = pl.program_id(0)
        c = c_ref[0]
        pT = pT_ref[0]
        aa = jnp.sum(c * c, axis=1, keepdims=True)
        bb = jnp.sum(pT * pT, axis=0, keepdims=True)
        d2 = aa + bb - 2.0 * jnp.dot(c, pT, preferred_element_type=jnp.float32)
        iota = jax.lax.broadcasted_iota(jnp.int32, (mt, P), 1)
        kiota = jax.lax.broadcasted_iota(jnp.int32, (mt, k), 1)

        def step(t, carry):
            d2c, idxb = carry
            m = jnp.min(d2c, axis=1, keepdims=True)
            amin = jnp.min(jnp.where(d2c == m, iota, P), axis=1, keepdims=True)
            idxb = jnp.where(kiota == t, amin, idxb)
            d2c = jnp.where(iota == amin, big, d2c)
            return d2c, idxb

        idx0 = jnp.zeros((mt, k), jnp.int32)
        _, idxb = jax.lax.fori_loop(0, k, step, (d2, idx0))
        out_ref[0] = idxb + b * P

    return pl.pallas_call(
        body,
        grid=(B, M // mt),
        in_specs=[
            pl.BlockSpec((1, mt, 8), lambda b, i: (b, i, 0)),
            pl.BlockSpec((1, 8, P), lambda b, i: (b, 0, 0)),
        ],
        out_specs=pl.BlockSpec((1, mt, k), lambda b, i: (b, i, 0)),
        out_shape=jax.ShapeDtypeStruct((B, M, k), jnp.int32),
    )(cxyz, pxyzT)


def _sc_gather_max(y2d, idx, k):
    BM, kk = idx.shape
    assert kk == k
    C = y2d.shape[1]
    kc = k
    while kc * C * 4 > 131072:
        kc //= 2
    nck = k // kc
    info = plsc.get_sparse_core_info()
    NW = info.num_cores * info.num_subcores
    npw = BM // NW
    total = npw * nck
    NG = C // 128
    idx2 = idx.reshape(BM * nck, kc)
    mesh = plsc.VectorSubcoreMesh(core_axis_name="c", subcore_axis_name="s")

    @functools.partial(
        pl.kernel,
        mesh=mesh,
        out_type=jax.ShapeDtypeStruct((BM, C), jnp.float32),
        scratch_types=[
            pltpu.VMEM((total, kc), jnp.int32),
            pltpu.VMEM((2, kc, C), jnp.float32),
            pltpu.VMEM((npw, C), jnp.float32),
            pltpu.SemaphoreType.DMA,
            pltpu.SemaphoreType.DMA,
        ],
    )
    def kfn(y_hbm, idx_hbm, out_hbm, idx_v, rows_v, out_v, sem_a, sem_b):
        cid = lax.axis_index("c")
        sid = lax.axis_index("s")
        wid = sid * info.num_cores + cid
        base = wid * total
        pltpu.sync_copy(idx_hbm.at[pl.ds(base, total)], idx_v)
        sems = [sem_a, sem_b]
        minus_inf = jnp.full((16,), -float('inf'), jnp.float32)

        def initbody(i2, carry):
            for gj in range(C // 16):
                out_v[i2, pl.ds(gj * 16, 16)] = minus_inf
            return carry

        lax.fori_loop(0, npw, initbody, 0)

        def fire(t, b):
            pltpu.async_copy(y_hbm.at[idx_v.at[t]], rows_v.at[b], sems[b])

        def waitslot(t, b):
            pltpu.make_async_copy(
                y_hbm.at[idx_v.at[t]], rows_v.at[b], sems[b]).wait()

        def reduce(t, b):
            i = t // nck if nck > 1 else t
            for g in range(NG):
                def rbody(r, acc):
                    return tuple(
                        jnp.maximum(acc[j],
                                    rows_v[b, r, pl.ds(g * 128 + j * 16, 16)])
                        for j in range(8))
                acc0 = tuple(minus_inf for _ in range(8))
                acc = lax.fori_loop(0, kc, rbody, acc0)
                for j in range(8):
                    sl = pl.ds(g * 128 + j * 16, 16)
                    out_v[i, sl] = jnp.maximum(out_v[i, sl], acc[j])

        fire(0, 0)
        fire(1, 1)

        def lbody(it, carry):
            t0 = it * 2
            for b in range(2):
                t = t0 + b
                waitslot(t, b)
                reduce(t, b)

                @pl.when(t + 2 < total)
                def _():
                    fire(t + 2, b)
            return carry

        lax.fori_loop(0, total // 2, lbody, 0)
        pltpu.sync_copy(out_v, out_hbm.at[pl.ds(wid * npw, npw)])

    return kfn(y2d, idx2)


def _interp(x1, x2T, f2, nt):
    B, N1, _ = x1.shape
    P2 = x2T.shape[2]
    C2 = f2.shape[2]
    big = float('inf')

    def body(x1_ref, x2T_ref, f2_ref, out_ref):
        c = x1_ref[0]
        pT = x2T_ref[0]
        fv = f2_ref[0]
        aa = jnp.sum(c * c, axis=1, keepdims=True)
        bb = jnp.sum(pT * pT, axis=0, keepdims=True)
        d2 = aa + bb - 2.0 * jnp.dot(c, pT, preferred_element_type=jnp.float32)
        d2 = jnp.maximum(d2, 0.0)
        d = jnp.where(d2 > 1e-12, jnp.sqrt(jnp.where(d2 > 1e-12, d2, 1.0)), 0.0)
        d = jnp.maximum(d, 1e-8)
        iota = jax.lax.broadcasted_iota(jnp.int32, (nt, P2), 1)
        dwork = d
        sel = jnp.zeros((nt, P2), jnp.bool_)
        for _ in range(3):
            m = jnp.min(dwork, axis=1, keepdims=True)
            amin = jnp.min(jnp.where(dwork == m, iota, P2), axis=1,
                           keepdims=True)
            onehot = iota == amin
            sel = jnp.logical_or(sel, onehot)
            dwork = jnp.where(onehot, big, dwork)
        w = jnp.where(sel, 1.0 / d, 0.0)
        w = w / jnp.sum(w, axis=1, keepdims=True)
        out_ref[0] = jnp.dot(w, fv, preferred_element_type=jnp.float32)

    return pl.pallas_call(
        body,
        grid=(B, N1 // nt),
        in_specs=[
            pl.BlockSpec((1, nt, 8), lambda b, i: (b, i, 0)),
            pl.BlockSpec((1, 8, P2), lambda b, i: (b, 0, 0)),
            pl.BlockSpec((1, P2, C2), lambda b, i: (b, 0, 0)),
        ],
        out_specs=pl.BlockSpec((1, nt, C2), lambda b, i: (b, i, 0)),
        out_shape=jax.ShapeDtypeStruct((B, N1, C2), jnp.float32),
    )(x1, x2T, f2)


def _centers_idx(P):
    M = max(1, P // 4)
    return jnp.linspace(0.0, P - 1, M).astype(jnp.int32)


def _prep_mlp(layers, cin_padded):
    out = []
    for i, l in enumerate(layers):
        Wt, b = _fold(l)
        if i == 0:
            Wt = _pad_rows(Wt) if Wt.shape[0] != cin_padded else Wt
        out.append((Wt, b))
    return out


def kernel(xyz, params):
    B, P0, _ = xyz.shape
    f32 = jnp.float32
    xyz = xyz.astype(f32)

    xyz_p = _pad_cols(xyz)
    xyzT = jnp.transpose(xyz_p, (0, 2, 1))

    sa1 = _prep_mlp(params['sa1'], 8)
    Y1 = _mlp(xyz_p.reshape(B * P0, 8), sa1, [True] * 3)
    idx1 = _centers_idx(P0)
    c1 = jnp.take(xyz_p, idx1, axis=1)
    nn1 = _knn_idx(c1, xyzT, k=32, mt=256)
    M1 = c1.shape[1]
    l1 = _sc_gather_max(Y1, nn1.reshape(B * M1, 32), 32)
    l1 = l1.reshape(B, M1, -1)

    P1 = c1.shape[1]
    c1T = jnp.transpose(c1, (0, 2, 1))
    in2 = _pad_cols(jnp.concatenate([c1[..., :3], l1], axis=-1))
    sa2 = _prep_mlp(params['sa2'], in2.shape[-1])
    Y2 = _mlp(in2.reshape(B * P1, -1), sa2, [True] * 3)
    idx2 = _centers_idx(P1)
    c2 = jnp.take(c1, idx2, axis=1)
    nn2 = _knn_idx(c2, c1T, k=64, mt=256)
    M2 = c2.shape[1]
    l2 = _sc_gather_max(Y2, nn2.reshape(B * M2, 64), 64)
    l2 = l2.reshape(B, M2, -1)

    P2 = c2.shape[1]
    c2T = jnp.transpose(c2, (0, 2, 1))
    in3 = _pad_cols(jnp.concatenate([c2[..., :3], l2], axis=-1))
    sa3 = _prep_mlp(params['sa3'], in3.shape[-1])
    Y3 = _mlp(in3.reshape(B * P2, -1), sa3, [True] * 3)
    idx3 = _centers_idx(P2)
    c3 = jnp.take(c2, idx3, axis=1)
    c3T = jnp.transpose(c3, (0, 2, 1))
    nn3 = _knn_idx(c3, c2T, k=128, mt=64)
    M3 = c3.shape[1]
    l3 = _sc_gather_max(Y3, nn3.reshape(B * M3, 128), 128)
    l3 = l3.reshape(B, M3, -1)

    i3 = _interp(c2, c3T, l3, nt=256)
    fp3 = _prep_mlp(params['fp3'], 1280)
    cat3 = jnp.concatenate([i3, l2], axis=-1).reshape(B * P2, -1)
    l2n = _mlp(cat3, fp3, [True] * 2).reshape(B, P2, -1)

    i2 = _interp(c1, c2T, l2n, nt=512)
    fp2 = _prep_mlp(params['fp2'], 384)
    cat2 = jnp.concatenate([i2, l1], axis=-1).reshape(B * P1, -1)
    l1n = _mlp(cat2, fp2, [True] * 2).reshape(B, P1, -1)

    i1 = _interp(xyz_p, c1T, l1n, nt=512)
    fp1 = _prep_mlp(params['fp1'], 128)
    l0n = _mlp(i1.reshape(B * P0, -1), fp1, [True] * 3)

    h = params['head']
    h1 = _fold({'W': h['c1W'], 'b': h['c1b'], 'g': h['g'], 'be': h['be'],
                'rm': h['rm'], 'rv': h['rv']})
    h2 = (h['c2W'].T, h['c2b'])
    out = _mlp(l0n, [h1, h2], [True, False])
    return out.reshape(B, P0, -1)


# speedup vs baseline: 8.2566x; 1.0354x over previous
import functools

import jax
import jax.numpy as jnp
from jax import lax
from jax.experimental import pallas as pl
from jax.experimental.pallas import tpu as pltpu
from jax.experimental.pallas import tpu_sc as plsc

_EPS = 1e-5


def _fold(layer):
    s = layer['g'] / jnp.sqrt(layer['rv'] + _EPS)
    Wt = (layer['W'] * s[:, None]).T
    b = s * (layer['b'] - layer['rm']) + layer['be']
    return Wt, b


def _pad_cols(x, m=8):
    c = x.shape[-1]
    p = (-c) % m
    if p == 0:
        return x
    return jnp.pad(x, [(0, 0)] * (x.ndim - 1) + [(0, p)])


def _pad_rows(w, m=8):
    r = w.shape[0]
    p = (-r) % m
    if p == 0:
        return w
    return jnp.pad(w, [(0, p), (0, 0)])


def _mlp(x, layers, relu_flags, tile=512):
    N = x.shape[0]
    tile = min(tile, N)
    nl = len(layers)

    flat = []
    in_specs = [pl.BlockSpec((tile, x.shape[1]), lambda i: (i, 0))]
    for Wt, b in layers:
        b2 = jnp.tile(b[None, :], (8, 1))
        in_specs.append(pl.BlockSpec(Wt.shape, lambda i: (0, 0)))
        in_specs.append(pl.BlockSpec(b2.shape, lambda i: (0, 0)))
        flat += [Wt, b2]
    Co = layers[-1][0].shape[1]

    def body(x_ref, *refs):
        out_ref = refs[-1]
        h = x_ref[...]
        for i in range(nl):
            W = refs[2 * i][...]
            b = refs[2 * i + 1][0:1, :]
            h = jnp.dot(h, W, preferred_element_type=jnp.float32) + b
            if relu_flags[i]:
                h = jnp.maximum(h, 0.0)
        out_ref[...] = h

    return pl.pallas_call(
        body,
        grid=(N // tile,),
        in_specs=in_specs,
        out_specs=pl.BlockSpec((tile, Co), lambda i: (i, 0)),
        out_shape=jax.ShapeDtypeStruct((N, Co), jnp.float32),
    )(x, *flat)


def _knn_idx(cxyz, pxyzT, k, mt):
    B, M, _ = cxyz.shape
    P = pxyzT.shape[2]
    big = float('inf')

    def body(c_ref, pT_ref, out_ref):
        b = pl.program_id(0)
        c = c_ref[0]
        pT = pT_ref[0]
        aa = jnp.sum(c * c, axis=1, keepdims=True)
        bb = jnp.sum(pT * pT, axis=0, keepdims=True)
        d2 = aa + bb - 2.0 * jnp.dot(c, pT, preferred_element_type=jnp.float32)
        iota = jax.lax.broadcasted_iota(jnp.int32, (mt, P), 1)
        kiota = jax.lax.broadcasted_iota(jnp.int32, (mt, k), 1)

        def step(t, carry):
            d2c, idxb = carry
            m = jnp.min(d2c, axis=1, keepdims=True)
            eq = d2c == m
            amin = jnp.min(jnp.where(eq, iota, P), axis=1, keepdims=True)
            idxb = jnp.where(kiota == t, amin, idxb)
            d2c = jnp.where(eq, big, d2c)
            return d2c, idxb

        idx0 = jnp.zeros((mt, k), jnp.int32)
        _, idxb = jax.lax.fori_loop(0, k, step, (d2, idx0))
        out_ref[0] = idxb + b * P

    return pl.pallas_call(
        body,
        grid=(B, M // mt),
        in_specs=[
            pl.BlockSpec((1, mt, 8), lambda b, i: (b, i, 0)),
            pl.BlockSpec((1, 8, P), lambda b, i: (b, 0, 0)),
        ],
        out_specs=pl.BlockSpec((1, mt, k), lambda b, i: (b, i, 0)),
        out_shape=jax.ShapeDtypeStruct((B, M, k), jnp.int32),
    )(cxyz, pxyzT)


def _sc_gather_max(y2d, idx, k):
    BM, kk = idx.shape
    assert kk == k
    C = y2d.shape[1]
    kc = k
    while kc * C * 4 > 131072:
        kc //= 2
    nck = k // kc
    info = plsc.get_sparse_core_info()
    NW = info.num_cores * info.num_subcores
    npw = BM // NW
    total = npw * nck
    NG = C // 128
    idx2 = idx.reshape(BM * nck, kc)
    mesh = plsc.VectorSubcoreMesh(core_axis_name="c", subcore_axis_name="s")

    @functools.partial(
        pl.kernel,
        mesh=mesh,
        out_type=jax.ShapeDtypeStruct((BM, C), jnp.float32),
        scratch_types=[
            pltpu.VMEM((total, kc), jnp.int32),
            pltpu.VMEM((2, kc, C), jnp.float32),
            pltpu.VMEM((npw, C), jnp.float32),
            pltpu.SemaphoreType.DMA,
            pltpu.SemaphoreType.DMA,
        ],
    )
    def kfn(y_hbm, idx_hbm, out_hbm, idx_v, rows_v, out_v, sem_a, sem_b):
        cid = lax.axis_index("c")
        sid = lax.axis_index("s")
        wid = sid * info.num_cores + cid
        base = wid * total
        pltpu.sync_copy(idx_hbm.at[pl.ds(base, total)], idx_v)
        sems = [sem_a, sem_b]
        minus_inf = jnp.full((16,), -float('inf'), jnp.float32)

        def initbody(i2, carry):
            for gj in range(C // 16):
                out_v[i2, pl.ds(gj * 16, 16)] = minus_inf
            return carry

        lax.fori_loop(0, npw, initbody, 0)

        def fire(t, b):
            pltpu.async_copy(y_hbm.at[idx_v.at[t]], rows_v.at[b], sems[b])

        def waitslot(t, b):
            pltpu.make_async_copy(
                y_hbm.at[idx_v.at[t]], rows_v.at[b], sems[b]).wait()

        def reduce(t, b):
            i = t // nck if nck > 1 else t
            for g in range(NG):
                def rbody(r, acc):
                    return tuple(
                        jnp.maximum(acc[j],
                                    rows_v[b, r, pl.ds(g * 128 + j * 16, 16)])
                        for j in range(8))
                acc0 = tuple(minus_inf for _ in range(8))
                acc = lax.fori_loop(0, kc, rbody, acc0)
                for j in range(8):
                    sl = pl.ds(g * 128 + j * 16, 16)
                    out_v[i, sl] = jnp.maximum(out_v[i, sl], acc[j])

        fire(0, 0)
        fire(1, 1)

        def lbody(it, carry):
            t0 = it * 2
            for b in range(2):
                t = t0 + b
                waitslot(t, b)
                reduce(t, b)

                @pl.when(t + 2 < total)
                def _():
                    fire(t + 2, b)
            return carry

        lax.fori_loop(0, total // 2, lbody, 0)
        pltpu.sync_copy(out_v, out_hbm.at[pl.ds(wid * npw, npw)])

    return kfn(y2d, idx2)


def _interp(x1, x2T, f2, nt):
    B, N1, _ = x1.shape
    P2 = x2T.shape[2]
    C2 = f2.shape[2]
    big = float('inf')

    def body(x1_ref, x2T_ref, f2_ref, out_ref):
        c = x1_ref[0]
        pT = x2T_ref[0]
        fv = f2_ref[0]
        aa = jnp.sum(c * c, axis=1, keepdims=True)
        bb = jnp.sum(pT * pT, axis=0, keepdims=True)
        d2 = aa + bb - 2.0 * jnp.dot(c, pT, preferred_element_type=jnp.float32)
        d2 = jnp.maximum(d2, 0.0)
        d = jnp.where(d2 > 1e-12, jnp.sqrt(jnp.where(d2 > 1e-12, d2, 1.0)), 0.0)
        d = jnp.maximum(d, 1e-8)
        iota = jax.lax.broadcasted_iota(jnp.int32, (nt, P2), 1)
        dwork = d
        sel = jnp.zeros((nt, P2), jnp.bool_)
        for _ in range(3):
            m = jnp.min(dwork, axis=1, keepdims=True)
            amin = jnp.min(jnp.where(dwork == m, iota, P2), axis=1,
                           keepdims=True)
            onehot = iota == amin
            sel = jnp.logical_or(sel, onehot)
            dwork = jnp.where(onehot, big, dwork)
        w = jnp.where(sel, 1.0 / d, 0.0)
        w = w / jnp.sum(w, axis=1, keepdims=True)
        out_ref[0] = jnp.dot(w, fv, preferred_element_type=jnp.float32)

    return pl.pallas_call(
        body,
        grid=(B, N1 // nt),
        in_specs=[
            pl.BlockSpec((1, nt, 8), lambda b, i: (b, i, 0)),
            pl.BlockSpec((1, 8, P2), lambda b, i: (b, 0, 0)),
            pl.BlockSpec((1, P2, C2), lambda b, i: (b, 0, 0)),
        ],
        out_specs=pl.BlockSpec((1, nt, C2), lambda b, i: (b, i, 0)),
        out_shape=jax.ShapeDtypeStruct((B, N1, C2), jnp.float32),
    )(x1, x2T, f2)


def _centers_idx(P):
    M = max(1, P // 4)
    return jnp.linspace(0.0, P - 1, M).astype(jnp.int32)


def _prep_mlp(layers, cin_padded):
    out = []
    for i, l in enumerate(layers):
        Wt, b = _fold(l)
        if i == 0:
            Wt = _pad_rows(Wt) if Wt.shape[0] != cin_padded else Wt
        out.append((Wt, b))
    return out


def kernel(xyz, params):
    B, P0, _ = xyz.shape
    f32 = jnp.float32
    xyz = xyz.astype(f32)

    xyz_p = _pad_cols(xyz)
    xyzT = jnp.transpose(xyz_p, (0, 2, 1))

    c1 = jnp.take(xyz_p, _centers_idx(P0), axis=1)
    P1 = c1.shape[1]
    c1T = jnp.transpose(c1, (0, 2, 1))
    c2 = jnp.take(c1, _centers_idx(P1), axis=1)
    P2 = c2.shape[1]
    c2T = jnp.transpose(c2, (0, 2, 1))
    c3 = jnp.take(c2, _centers_idx(P2), axis=1)
    c3T = jnp.transpose(c3, (0, 2, 1))
    M1, M2, M3 = c1.shape[1], c2.shape[1], c3.shape[1]

    sa1 = _prep_mlp(params['sa1'], 8)
    Y1 = _mlp(xyz_p.reshape(B * P0, 8), sa1, [True] * 3)
    nn1 = _knn_idx(c1, xyzT, k=32, mt=256)
    l1 = _sc_gather_max(Y1, nn1.reshape(B * M1, 32), 32)
    nn2 = _knn_idx(c2, c1T, k=64, mt=256)
    nn3 = _knn_idx(c3, c2T, k=128, mt=64)
    l1 = l1.reshape(B, M1, -1)

    in2 = _pad_cols(jnp.concatenate([c1[..., :3], l1], axis=-1))
    sa2 = _prep_mlp(params['sa2'], in2.shape[-1])
    Y2 = _mlp(in2.reshape(B * P1, -1), sa2, [True] * 3)
    l2 = _sc_gather_max(Y2, nn2.reshape(B * M2, 64), 64)
    l2 = l2.reshape(B, M2, -1)

    in3 = _pad_cols(jnp.concatenate([c2[..., :3], l2], axis=-1))
    sa3 = _prep_mlp(params['sa3'], in3.shape[-1])
    Y3 = _mlp(in3.reshape(B * P2, -1), sa3, [True] * 3)
    l3 = _sc_gather_max(Y3, nn3.reshape(B * M3, 128), 128)
    l3 = l3.reshape(B, M3, -1)

    i3 = _interp(c2, c3T, l3, nt=256)
    fp3 = _prep_mlp(params['fp3'], 1280)
    cat3 = jnp.concatenate([i3, l2], axis=-1).reshape(B * P2, -1)
    l2n = _mlp(cat3, fp3, [True] * 2).reshape(B, P2, -1)

    i2 = _interp(c1, c2T, l2n, nt=512)
    fp2 = _prep_mlp(params['fp2'], 384)
    cat2 = jnp.concatenate([i2, l1], axis=-1).reshape(B * P1, -1)
    l1n = _mlp(cat2, fp2, [True] * 2).reshape(B, P1, -1)

    i1 = _interp(xyz_p, c1T, l1n, nt=512)
    fp1 = _prep_mlp(params['fp1'], 128)
    l0n = _mlp(i1.reshape(B * P0, -1), fp1, [True] * 3)

    h = params['head']
    h1 = _fold({'W': h['c1W'], 'b': h['c1b'], 'g': h['g'], 'be': h['be'],
                'rm': h['rm'], 'rv': h['rv']})
    h2 = (h['c2W'].T, h['c2b'])
    out = _mlp(l0n, [h1, h2], [True, False])
    return out.reshape(B, P0, -1)


# speedup vs baseline: 8.4303x; 1.0210x over previous
import functools

import jax
import jax.numpy as jnp
from jax import lax
from jax.experimental import pallas as pl
from jax.experimental.pallas import tpu as pltpu
from jax.experimental.pallas import tpu_sc as plsc

_EPS = 1e-5


def _fold(layer):
    s = layer['g'] / jnp.sqrt(layer['rv'] + _EPS)
    Wt = (layer['W'] * s[:, None]).T
    b = s * (layer['b'] - layer['rm']) + layer['be']
    return Wt, b


def _pad_cols(x, m=8):
    c = x.shape[-1]
    p = (-c) % m
    if p == 0:
        return x
    return jnp.pad(x, [(0, 0)] * (x.ndim - 1) + [(0, p)])


def _pad_rows(w, m=8):
    r = w.shape[0]
    p = (-r) % m
    if p == 0:
        return w
    return jnp.pad(w, [(0, p), (0, 0)])


def _mlp(x, layers, relu_flags, tile=512):
    N = x.shape[0]
    tile = min(tile, N)
    nl = len(layers)

    flat = []
    in_specs = [pl.BlockSpec((tile, x.shape[1]), lambda i: (i, 0))]
    for Wt, b in layers:
        b2 = jnp.tile(b[None, :], (8, 1))
        in_specs.append(pl.BlockSpec(Wt.shape, lambda i: (0, 0)))
        in_specs.append(pl.BlockSpec(b2.shape, lambda i: (0, 0)))
        flat += [Wt, b2]
    Co = layers[-1][0].shape[1]

    def body(x_ref, *refs):
        out_ref = refs[-1]
        h = x_ref[...]
        for i in range(nl):
            W = refs[2 * i][...]
            b = refs[2 * i + 1][0:1, :]
            h = jnp.dot(h, W, preferred_element_type=jnp.float32) + b
            if relu_flags[i]:
                h = jnp.maximum(h, 0.0)
        out_ref[...] = h

    return pl.pallas_call(
        body,
        grid=(N // tile,),
        in_specs=in_specs,
        out_specs=pl.BlockSpec((tile, Co), lambda i: (i, 0)),
        out_shape=jax.ShapeDtypeStruct((N, Co), jnp.float32),
    )(x, *flat)


def _knn_idx(cxyz, pxyzT, k, mt):
    B, M, _ = cxyz.shape
    P = pxyzT.shape[2]
    big = float('inf')

    def body(c_ref, pT_ref, out_ref):
        b = pl.program_id(0)
        c = c_ref[0]
        pT = pT_ref[0]
        aa = jnp.sum(c * c, axis=1, keepdims=True)
        bb = jnp.sum(pT * pT, axis=0, keepdims=True)
        d2 = aa + bb - 2.0 * jnp.dot(c, pT, preferred_element_type=jnp.float32)
        iota = jax.lax.broadcasted_iota(jnp.int32, (mt, P), 1)
        kiota = jax.lax.broadcasted_iota(jnp.int32, (mt, k), 1)

        def step(t, carry):
            d2c, idxb = carry
            m = jnp.min(d2c, axis=1, keepdims=True)
            eq = d2c == m
            amin = jnp.min(jnp.where(eq, iota, P), axis=1, keepdims=True)
            idxb = jnp.where(kiota == t, amin, idxb)
            d2c = jnp.where(eq, big, d2c)
            return d2c, idxb

        idx0 = jnp.zeros((mt, k), jnp.int32)
        _, idxb = jax.lax.fori_loop(0, k, step, (d2, idx0))
        out_ref[0] = idxb + b * P

    return pl.pallas_call(
        body,
        grid=(B, M // mt),
        in_specs=[
            pl.BlockSpec((1, mt, 8), lambda b, i: (b, i, 0)),
            pl.BlockSpec((1, 8, P), lambda b, i: (b, 0, 0)),
        ],
        out_specs=pl.BlockSpec((1, mt, k), lambda b, i: (b, i, 0)),
        out_shape=jax.ShapeDtypeStruct((B, M, k), jnp.int32),
    )(cxyz, pxyzT)


def _sc_gather_max(y2d, idx, k):
    BM, kk = idx.shape
    assert kk == k
    C = y2d.shape[1]
    kc = k
    while kc * C * 4 > 131072:
        kc //= 2
    nck = k // kc
    info = plsc.get_sparse_core_info()
    NW = info.num_cores * info.num_subcores
    npw = BM // NW
    total = npw * nck
    NG = C // 128
    idx2 = idx.reshape(BM * nck, kc)
    mesh = plsc.VectorSubcoreMesh(core_axis_name="c", subcore_axis_name="s")

    @functools.partial(
        pl.kernel,
        mesh=mesh,
        out_type=jax.ShapeDtypeStruct((BM, C), jnp.float32),
        scratch_types=[
            pltpu.VMEM((total, kc), jnp.int32),
            pltpu.VMEM((2, kc, C), jnp.float32),
            pltpu.VMEM((npw, C), jnp.float32),
            pltpu.SemaphoreType.DMA,
            pltpu.SemaphoreType.DMA,
        ],
    )
    def kfn(y_hbm, idx_hbm, out_hbm, idx_v, rows_v, out_v, sem_a, sem_b):
        cid = lax.axis_index("c")
        sid = lax.axis_index("s")
        wid = sid * info.num_cores + cid
        base = wid * total
        pltpu.sync_copy(idx_hbm.at[pl.ds(base, total)], idx_v)
        sems = [sem_a, sem_b]
        minus_inf = jnp.full((16,), -float('inf'), jnp.float32)

        def initbody(i2, carry):
            for gj in range(C // 16):
                out_v[i2, pl.ds(gj * 16, 16)] = minus_inf
            return carry

        lax.fori_loop(0, npw, initbody, 0)

        def fire(t, b):
            pltpu.async_copy(y_hbm.at[idx_v.at[t]], rows_v.at[b], sems[b])

        def waitslot(t, b):
            pltpu.make_async_copy(
                y_hbm.at[idx_v.at[t]], rows_v.at[b], sems[b]).wait()

        def reduce(t, b):
            i = t // nck if nck > 1 else t
            for g in range(NG):
                def rbody(r, acc):
                    return tuple(
                        jnp.maximum(acc[j],
                                    rows_v[b, r, pl.ds(g * 128 + j * 16, 16)])
                        for j in range(8))
                acc0 = tuple(minus_inf for _ in range(8))
                acc = lax.fori_loop(0, kc, rbody, acc0)
                for j in range(8):
                    sl = pl.ds(g * 128 + j * 16, 16)
                    out_v[i, sl] = jnp.maximum(out_v[i, sl], acc[j])

        fire(0, 0)
        fire(1, 1)

        def lbody(it, carry):
            t0 = it * 2
            for b in range(2):
                t = t0 + b
                waitslot(t, b)
                reduce(t, b)

                @pl.when(t + 2 < total)
                def _():
                    fire(t + 2, b)
            return carry

        lax.fori_loop(0, total // 2, lbody, 0)
        pltpu.sync_copy(out_v, out_hbm.at[pl.ds(wid * npw, npw)])

    return kfn(y2d, idx2)


def _interp(x1, x2T, f2, nt):
    B, N1, _ = x1.shape
    P2 = x2T.shape[2]
    C2 = f2.shape[2]
    big = float('inf')

    def body(x1_ref, x2T_ref, f2_ref, out_ref):
        c = x1_ref[0]
        pT = x2T_ref[0]
        fv = f2_ref[0]
        aa = jnp.sum(c * c, axis=1, keepdims=True)
        bb = jnp.sum(pT * pT, axis=0, keepdims=True)
        d2 = aa + bb - 2.0 * jnp.dot(c, pT, preferred_element_type=jnp.float32)
        d2 = jnp.maximum(d2, 0.0)
        d = jnp.where(d2 > 1e-12, jnp.sqrt(jnp.where(d2 > 1e-12, d2, 1.0)), 0.0)
        d = jnp.maximum(d, 1e-8)
        iota = jax.lax.broadcasted_iota(jnp.int32, (nt, P2), 1)
        dwork = d
        sel = jnp.zeros((nt, P2), jnp.bool_)
        for _ in range(3):
            m = jnp.min(dwork, axis=1, keepdims=True)
            amin = jnp.min(jnp.where(dwork == m, iota, P2), axis=1,
                           keepdims=True)
            onehot = iota == amin
            sel = jnp.logical_or(sel, onehot)
            dwork = jnp.where(onehot, big, dwork)
        w = jnp.where(sel, 1.0 / d, 0.0)
        w = w / jnp.sum(w, axis=1, keepdims=True)
        out_ref[0] = jnp.dot(w, fv, preferred_element_type=jnp.float32)

    return pl.pallas_call(
        body,
        grid=(B, N1 // nt),
        in_specs=[
            pl.BlockSpec((1, nt, 8), lambda b, i: (b, i, 0)),
            pl.BlockSpec((1, 8, P2), lambda b, i: (b, 0, 0)),
            pl.BlockSpec((1, P2, C2), lambda b, i: (b, 0, 0)),
        ],
        out_specs=pl.BlockSpec((1, nt, C2), lambda b, i: (b, i, 0)),
        out_shape=jax.ShapeDtypeStruct((B, N1, C2), jnp.float32),
    )(x1, x2T, f2)


def _centers_idx(P):
    M = max(1, P // 4)
    return jnp.linspace(0.0, P - 1, M).astype(jnp.int32)


def _prep_mlp(layers, cin_padded):
    out = []
    for i, l in enumerate(layers):
        Wt, b = _fold(l)
        if i == 0:
            Wt = _pad_rows(Wt) if Wt.shape[0] != cin_padded else Wt
        out.append((Wt, b))
    return out


def kernel(xyz, params):
    B, P0, _ = xyz.shape
    f32 = jnp.float32
    xyz = xyz.astype(f32)

    xyz_p = _pad_cols(xyz)
    xyzT = jnp.transpose(xyz_p, (0, 2, 1))

    c1 = jnp.take(xyz_p, _centers_idx(P0), axis=1)
    P1 = c1.shape[1]
    c1T = jnp.transpose(c1, (0, 2, 1))
    c2 = jnp.take(c1, _centers_idx(P1), axis=1)
    P2 = c2.shape[1]
    c2T = jnp.transpose(c2, (0, 2, 1))
    c3 = jnp.take(c2, _centers_idx(P2), axis=1)
    c3T = jnp.transpose(c3, (0, 2, 1))
    M1, M2, M3 = c1.shape[1], c2.shape[1], c3.shape[1]

    sa1 = _prep_mlp(params['sa1'], 8)
    Y1 = _mlp(xyz_p.reshape(B * P0, 8), sa1, [True] * 3)
    nn1 = _knn_idx(c1, xyzT, k=32, mt=512)
    l1 = _sc_gather_max(Y1, nn1.reshape(B * M1, 32), 32)
    nn2 = _knn_idx(c2, c1T, k=64, mt=256)
    nn3 = _knn_idx(c3, c2T, k=128, mt=64)
    l1 = l1.reshape(B, M1, -1)

    in2 = _pad_cols(jnp.concatenate([c1[..., :3], l1], axis=-1))
    sa2 = _prep_mlp(params['sa2'], in2.shape[-1])
    Y2 = _mlp(in2.reshape(B * P1, -1), sa2, [True] * 3)
    l2 = _sc_gather_max(Y2, nn2.reshape(B * M2, 64), 64)
    l2 = l2.reshape(B, M2, -1)

    in3 = _pad_cols(jnp.concatenate([c2[..., :3], l2], axis=-1))
    sa3 = _prep_mlp(params['sa3'], in3.shape[-1])
    Y3 = _mlp(in3.reshape(B * P2, -1), sa3, [True] * 3)
    l3 = _sc_gather_max(Y3, nn3.reshape(B * M3, 128), 128)
    l3 = l3.reshape(B, M3, -1)

    i3 = _interp(c2, c3T, l3, nt=256)
    fp3 = _prep_mlp(params['fp3'], 1280)
    cat3 = jnp.concatenate([i3, l2], axis=-1).reshape(B * P2, -1)
    l2n = _mlp(cat3, fp3, [True] * 2).reshape(B, P2, -1)

    i2 = _interp(c1, c2T, l2n, nt=512)
    fp2 = _prep_mlp(params['fp2'], 384)
    cat2 = jnp.concatenate([i2, l1], axis=-1).reshape(B * P1, -1)
    l1n = _mlp(cat2, fp2, [True] * 2).reshape(B, P1, -1)

    i1 = _interp(xyz_p, c1T, l1n, nt=512)
    fp1 = _prep_mlp(params['fp1'], 128)
    l0n = _mlp(i1.reshape(B * P0, -1), fp1, [True] * 3)

    h = params['head']
    h1 = _fold({'W': h['c1W'], 'b': h['c1b'], 'g': h['g'], 'be': h['be'],
                'rm': h['rm'], 'rv': h['rv']})
    h2 = (h['c2W'].T, h['c2b'])
    out = _mlp(l0n, [h1, h2], [True, False])
    return out.reshape(B, P0, -1)
